# baseline (speedup 1.0000x reference)
import jax
import jax.numpy as jnp
from jax.experimental import pallas as pl

NUM_NODES = 100000


def _extract_features(pos, faces):
    v0 = pos[faces[:, 0]]
    v1 = pos[faces[:, 1]]
    v2 = pos[faces[:, 2]]
    e1 = v1 - v0
    e2 = v2 - v0
    n = jnp.cross(e1, e2)
    nrm = jnp.linalg.norm(n, axis=1, keepdims=True)
    area = 0.5 * nrm
    normal = n / (nrm + 1e-8)
    centroid = (v0 + v1 + v2) / 3.0
    return jnp.concatenate([centroid, normal, area], axis=1)


def _face_conv(faces, feats, W_self, W_nb, b):
    v_idx = faces.reshape(-1)
    f_rep = jnp.repeat(feats, 3, axis=0)
    sums = jax.ops.segment_sum(f_rep, v_idx, num_segments=NUM_NODES)
    cnt = jax.ops.segment_sum(jnp.ones((v_idx.shape[0],), feats.dtype),
                              v_idx, num_segments=NUM_NODES)
    vfeat = sums / jnp.maximum(cnt, 1.0)[:, None]
    agg = vfeat[faces].mean(axis=1)
    return feats @ W_self + agg @ W_nb + b


def _copy_kernel(x_ref, o_ref):
    o_ref[...] = x_ref[...]


def kernel(pos, faces, W1s, W1n, b1, W2s, W2n, b2, W3s, W3n, b3,
           W4s, W4n, b4):
    feats = jax.nn.relu(_face_conv(faces, _extract_features(pos, faces),
                                   W1s, W1n, b1))
    F2 = faces.shape[0] // 2
    faces, feats = faces[:F2], 0.5 * (feats[:F2] + feats[F2:2 * F2])
    feats = jax.nn.relu(_face_conv(faces, feats, W2s, W2n, b2))
    F2 = faces.shape[0] // 2
    faces, feats = faces[:F2], 0.5 * (feats[:F2] + feats[F2:2 * F2])
    feats = jax.nn.relu(_face_conv(faces, feats, W3s, W3n, b3))
    F2 = faces.shape[0] // 2
    faces, feats = faces[:F2], 0.5 * (feats[:F2] + feats[F2:2 * F2])
    feats = _face_conv(faces, feats, W4s, W4n, b4)
    feats = pl.pallas_call(
        _copy_kernel,
        out_shape=jax.ShapeDtypeStruct(feats.shape, feats.dtype),
    )(feats)
    return (faces, feats)


# speedup vs baseline: 3.2516x; 3.2516x over previous
import functools

import jax
import jax.numpy as jnp
from jax import lax
from jax.experimental import pallas as pl
from jax.experimental.pallas import tpu as pltpu
from jax.experimental.pallas import tpu_sc as plsc

V = 100000
VPAD = 100352
DUMMY = 100000
STRIPE = VPAD // 16
F_REAL = (200000, 100000, 50000, 25000)
F_PAD = (200704, 100352, 50176, 25600)
NC, NS = 2, 16
NW = NC * NS
BF = 1000
BV = 512

_SC_PARAMS = pltpu.CompilerParams(use_tc_tiling_on_sc=False)

_mesh = functools.partial(
    plsc.VectorSubcoreMesh,
    core_axis_name="c", subcore_axis_name="s", num_cores=NC, num_subcores=NS)


def _sc_gather(table, idx3, fp, d):
  m_tot = fp // 1024

  def body(table_h, idx_h, out_h, ibuf, rows, sem):
    core = lax.axis_index("c")
    sub = lax.axis_index("s")
    wid = sub * NC + core
    cnt = (m_tot - wid + NW - 1) // NW
    for j in range(3):
      def mbody(m, _):
        mb = wid + m * NW
        base = mb * 1024
        row = mb * 8
        pltpu.sync_copy(idx_h.at[j, pl.ds(row, 8), :], ibuf)
        for r in range(8):
          pltpu.async_copy(table_h.at[ibuf.at[r]],
                           rows.at[pl.ds(128 * r, 128)], sem).wait()
        pltpu.sync_copy(rows, out_h.at[j, pl.ds(base, 1024), :])
        return 0
      lax.fori_loop(0, cnt, mbody, 0)

  return pl.kernel(
      body,
      out_type=jax.ShapeDtypeStruct((3, fp, d), jnp.float32),
      mesh=_mesh(),
      scratch_types=[
          pltpu.VMEM((8, 128), jnp.int32),
          pltpu.VMEM((1024, d), jnp.float32),
          pltpu.SemaphoreType.DMA,
      ],
      compiler_params=_SC_PARAMS,
  )(table, idx3)


def _sc_scatter(data, idx3, fp, nchunks, split):
  n_out = 2 if split else nchunks
  cpc = 1 if split else nchunks // 2
  m_tot = fp // 1024
  full = split

  def body(data_h, idx_h, out_h, acc, dbuf, ib0, ib1, ib2):
    core = lax.axis_index("c")
    sub = lax.axis_index("s")

    for cc in range(cpc):
      def zb(i, _):
        dbuf[i] = jnp.zeros((16,), jnp.float32)
        return 0
      lax.fori_loop(0, 1024, zb, 0)
      for t in range(6):
        pltpu.sync_copy(dbuf, acc.at[pl.ds(sub * STRIPE + t * 1024, 1024), :])
      pltpu.sync_copy(dbuf.at[pl.ds(0, 128)],
                      acc.at[pl.ds(sub * STRIPE + 6144, 128), :])
      plsc.subcore_barrier()

      if split:
        ch = 0
        m2 = m_tot // 2
        mb0 = core * m2 + sub
        cnt = (m2 - sub + NS - 1) // NS
      else:
        ch = core * cpc + cc
        mb0 = sub
        cnt = (m_tot - sub + NS - 1) // NS
      col = 16 * ch

      def mbody(m, _):
        mb = mb0 + m * NS
        base = mb * 1024
        row = mb * 8
        if full:
          pltpu.sync_copy(data_h.at[pl.ds(base, 1024), :], dbuf)
        else:
          pltpu.sync_copy(data_h.at[pl.ds(base, 1024), pl.ds(col, 16)], dbuf)
        pltpu.sync_copy(idx_h.at[0, pl.ds(row, 8), :], ib0)
        pltpu.sync_copy(idx_h.at[1, pl.ds(row, 8), :], ib1)
        pltpu.sync_copy(idx_h.at[2, pl.ds(row, 8), :], ib2)
        for ib in (ib0, ib1, ib2):
          for r in range(8):
            pltpu.sync_copy(dbuf.at[pl.ds(128 * r, 128)],
                            acc.at[ib.at[r]], add=True)
        return 0
      lax.fori_loop(0, cnt, mbody, 0)
      plsc.subcore_barrier()

      oi = core if split else ch
      pltpu.sync_copy(acc.at[pl.ds(sub * STRIPE, STRIPE), :],
                      out_h.at[oi, pl.ds(sub * STRIPE, STRIPE), :])
      plsc.subcore_barrier()

  return pl.kernel(
      body,
      out_type=jax.ShapeDtypeStruct((n_out, VPAD, 16), jnp.float32),
      mesh=_mesh(),
      scratch_types=[
          pltpu.VMEM_SHARED((VPAD, 16), jnp.float32),
          pltpu.VMEM((1024, 16), jnp.float32),
          pltpu.VMEM((8, 128), jnp.int32),
          pltpu.VMEM((8, 128), jnp.int32),
          pltpu.VMEM((8, 128), jnp.int32),
      ],
      compiler_params=_SC_PARAMS,
  )(data, idx3)


def _tc_features(gpos, fp1):
  nb = F_REAL[0] // BF

  def body(g0_r, g1_r, g2_r, o_r):
    v0, v1, v2 = g0_r[0], g1_r[0], g2_r[0]
    c = lambda v, k: v[:, k:k + 1]
    e1 = [c(v1, k) - c(v0, k) for k in range(3)]
    e2 = [c(v2, k) - c(v0, k) for k in range(3)]
    nx = e1[1] * e2[2] - e1[2] * e2[1]
    ny = e1[2] * e2[0] - e1[0] * e2[2]
    nz = e1[0] * e2[1] - e1[1] * e2[0]
    nrm = jnp.sqrt(nx * nx + ny * ny + nz * nz)
    area = 0.5 * nrm
    inv = 1.0 / (nrm + 1e-8)
    cent = [(c(v0, k) + c(v1, k) + c(v2, k)) * (1.0 / 3.0) for k in range(3)]
    fid = (pl.program_id(0) * BF
           + lax.broadcasted_iota(jnp.int32, (BF, 1), 0))
    one = jnp.ones((BF, 1), jnp.float32)
    ind = [one] + [(fid < F_REAL[l]).astype(jnp.float32) for l in (1, 2, 3)]
    zero5 = jnp.zeros((BF, 5), jnp.float32)
    o_r[...] = jnp.concatenate(
        cent + [nx * inv, ny * inv, nz * inv, area] + ind + [zero5], axis=1)

  gspec = lambda j: pl.BlockSpec((1, BF, 16), lambda i, j=j: (j, i, 0))
  return pl.pallas_call(
      body,
      grid=(nb,),
      in_specs=[gspec(0), gspec(1), gspec(2)],
      out_specs=pl.BlockSpec((BF, 16), lambda i: (i, 0)),
      out_shape=jax.ShapeDtypeStruct((fp1, 16), jnp.float32),
  )(gpos, gpos, gpos)


def _tc_vertex1(sums1):
  nb = VPAD // BV

  def body(sa_r, sb_r, vt_r, ic_r):
    s = sa_r[0] + sb_r[0]
    inv4 = 1.0 / jnp.maximum(s[:, 7:11], 1.0)
    vt_r[...] = s * inv4[:, 0:1]
    ic_r[...] = jnp.concatenate(
        [inv4, jnp.zeros((BV, 12), jnp.float32)], axis=1)

  pspec = lambda p: pl.BlockSpec((1, BV, 16), lambda i, p=p: (p, i, 0))
  return pl.pallas_call(
      body,
      grid=(nb,),
      in_specs=[pspec(0), pspec(1)],
      out_specs=[pl.BlockSpec((BV, 16), lambda i: (i, 0))] * 2,
      out_shape=[jax.ShapeDtypeStruct((VPAD, 16), jnp.float32)] * 2,
  )(sums1, sums1)


def _tc_vertex(sums, invc, wn, lcol, dout):
  nb = VPAD // BV

  def body(s_r, ic_r, w_r, o_r):
    ci = pl.program_id(1)
    p = jnp.dot(s_r[0], w_r[...], preferred_element_type=jnp.float32)
    acc = jnp.where(ci == 0, p, o_r[...] + p)
    inv = ic_r[:, lcol:lcol + 1]
    o_r[...] = jnp.where(ci == 3, acc * inv, acc)

  return pl.pallas_call(
      body,
      grid=(nb, 4),
      in_specs=[
          pl.BlockSpec((1, BV, 16), lambda i, c: (c, i, 0)),
          pl.BlockSpec((BV, 16), lambda i, c: (i, 0)),
          pl.BlockSpec((16, dout), lambda i, c: (c, 0)),
      ],
      out_specs=pl.BlockSpec((BV, dout), lambda i, c: (i, 0)),
      out_shape=jax.ShapeDtypeStruct((VPAD, dout), jnp.float32),
  )(sums, invc, wn)


def _tc_face(feats, g, ws, b, f_real, dfe, dg, dout, fp_out, wn=None,
             pool=True, relu=True):
  nb = (f_real // 2 if pool else f_real) // BF

  def one_half(fe, g0, g1, g2, w, wnv, bv):
    agg = (g0 + g1 + g2) * (1.0 / 3.0)
    z = jnp.dot(fe, w, preferred_element_type=jnp.float32)
    if wnv is not None:
      z = z + jnp.dot(agg, wnv, preferred_element_type=jnp.float32)
    else:
      z = z + agg
    z = z + bv
    return jnp.maximum(z, 0.0) if relu else z

  def body(*refs):
    if pool:
      (fl, fh, g0l, g1l, g2l, g0h, g1h, g2h, w_r, *rest) = refs
    else:
      (fl, g0l, g1l, g2l, w_r, *rest) = refs
    if wn is not None:
      wn_r = rest[0]
      rest = rest[1:]
    b_r, o_r = rest
    wv = w_r[...]
    wnv = wn_r[...] if wn is not None else None
    bv = b_r[...]
    zl = one_half(fl[...], g0l[0], g1l[0], g2l[0], wv, wnv, bv)
    if pool:
      zh = one_half(fh[...], g0h[0], g1h[0], g2h[0], wv, wnv, bv)
      o_r[...] = 0.5 * (zl + zh)
    else:
      o_r[...] = zl

  fspec = lambda off: pl.BlockSpec((BF, dfe), lambda i, off=off: (i + off, 0))
  gspec = lambda j, off: pl.BlockSpec(
      (1, BF, dg), lambda i, j=j, off=off: (j, i + off, 0))
  in_specs = [fspec(0)]
  args = [feats]
  if pool:
    in_specs.append(fspec(nb))
    args.append(feats)
  in_specs += [gspec(j, 0) for j in range(3)]
  args += [g, g, g]
  if pool:
    in_specs += [gspec(j, nb) for j in range(3)]
    args += [g, g, g]
  in_specs.append(pl.BlockSpec((dfe, dout), lambda i: (0, 0)))
  args.append(ws)
  if wn is not None:
    in_specs.append(pl.BlockSpec((dg, dout), lambda i: (0, 0)))
    args.append(wn)
  in_specs.append(pl.BlockSpec((1, dout), lambda i: (0, 0)))
  args.append(b.reshape(1, dout))

  return pl.pallas_call(
      body,
      grid=(nb,),
      in_specs=in_specs,
      out_specs=pl.BlockSpec((BF, dout), lambda i: (i, 0)),
      out_shape=jax.ShapeDtypeStruct((fp_out, dout), jnp.float32),
  )(*args)


def kernel(pos, faces, W1s, W1n, b1, W2s, W2n, b2, W3s, W3n, b3,
           W4s, W4n, b4):
  facesT = faces.T.astype(jnp.int32)
  idx = []
  for l in range(4):
    a = jnp.concatenate(
        [facesT[:, :F_REAL[l]],
         jnp.full((3, F_PAD[l] - F_REAL[l]), DUMMY, jnp.int32)], axis=1)
    idx.append(a.reshape(3, F_PAD[l] // 128, 128))

  pos16 = jnp.pad(pos.astype(jnp.float32), ((0, VPAD - V), (0, 13)))
  w1s16 = jnp.pad(W1s, ((0, 9), (0, 0)))
  w1n16 = jnp.pad(W1n, ((0, 9), (0, 0)))

  gpos = _sc_gather(pos16, idx[0], F_PAD[0], 16)
  feats1 = _tc_features(gpos, F_PAD[0])
  sums1 = _sc_scatter(feats1, idx[0], F_PAD[0], 1, split=True)
  vt1, invc = _tc_vertex1(sums1)
  g1 = _sc_gather(vt1, idx[0], F_PAD[0], 16)
  feats2 = _tc_face(feats1, g1, w1s16, b1, F_REAL[0], 16, 16, 64,
                    F_PAD[1], wn=w1n16, pool=True, relu=True)

  sums2 = _sc_scatter(feats2, idx[1], F_PAD[1], 4, split=False)
  vt2 = _tc_vertex(sums2, invc, W2n, 1, 64)
  g2 = _sc_gather(vt2, idx[1], F_PAD[1], 64)
  feats3 = _tc_face(feats2, g2, W2s, b2, F_REAL[1], 64, 64, 64,
                    F_PAD[2], pool=True, relu=True)

  sums3 = _sc_scatter(feats3, idx[2], F_PAD[2], 4, split=False)
  vt3 = _tc_vertex(sums3, invc, W3n, 2, 64)
  g3 = _sc_gather(vt3, idx[2], F_PAD[2], 64)
  feats4 = _tc_face(feats3, g3, W3s, b3, F_REAL[2], 64, 64, 64,
                    F_PAD[3], pool=True, relu=True)

  sums4 = _sc_scatter(feats4, idx[3], F_PAD[3], 4, split=False)
  vt4 = _tc_vertex(sums4, invc, W4n, 3, 16)
  g4 = _sc_gather(vt4, idx[3], F_PAD[3], 16)
  out = _tc_face(feats4, g4, W4s, b4, F_REAL[3], 64, 16, 16,
                 F_REAL[3], pool=False, relu=False)

  return (faces[:F_REAL[3]], out)


# speedup vs baseline: 6.7516x; 2.0764x over previous
import functools

import jax
import jax.numpy as jnp
from jax import lax
from jax.experimental import pallas as pl
from jax.experimental.pallas import tpu as pltpu
from jax.experimental.pallas import tpu_sc as plsc

V = 100000
VPAD = 100352
DUMMY = 100000
STRIPE = VPAD // 16
F_REAL = (200000, 100000, 50000, 25000)
F_PAD = (200704, 100352, 50176, 25600)
NC, NS = 2, 16
NW = NC * NS
BF = 1000
BV = 512

_SC_PARAMS = pltpu.CompilerParams(use_tc_tiling_on_sc=False)

_mesh = functools.partial(
    plsc.VectorSubcoreMesh,
    core_axis_name="c", subcore_axis_name="s", num_cores=NC, num_subcores=NS)


def _sc_gather(table, idx3, fp, d):
  m_tot = fp // 1024

  def body(table_h, idx_h, out_h, ibuf, rows, sem):
    core = lax.axis_index("c")
    sub = lax.axis_index("s")
    wid = sub * NC + core
    cnt = (m_tot - wid + NW - 1) // NW
    for j in range(3):
      def mbody(m, _):
        mb = wid + m * NW
        base = mb * 1024
        row = mb * 8
        pltpu.sync_copy(idx_h.at[j, pl.ds(row, 8), :], ibuf)
        for r in range(8):
          pltpu.async_copy(table_h.at[ibuf.at[r]],
                           rows.at[pl.ds(128 * r, 128)], sem).wait()
        pltpu.sync_copy(rows, out_h.at[j, pl.ds(base, 1024), :])
        return 0
      lax.fori_loop(0, cnt, mbody, 0)

  return pl.kernel(
      body,
      out_type=jax.ShapeDtypeStruct((3, fp, d), jnp.float32),
      mesh=_mesh(),
      scratch_types=[
          pltpu.VMEM((8, 128), jnp.int32),
          pltpu.VMEM((1024, d), jnp.float32),
          pltpu.SemaphoreType.DMA,
      ],
      compiler_params=_SC_PARAMS,
  )(table, idx3)


def _sc_scatter(data, idx3, fp, nchunks, split):
  n_out = 2 if split else nchunks
  cpc = 1 if split else nchunks // 2
  m_tot = fp // 1024
  full = split

  def body(data_h, idx_h, out_h, acc, dbuf, ib0, ib1, ib2):
    core = lax.axis_index("c")
    sub = lax.axis_index("s")

    for cc in range(cpc):
      def zb(i, _):
        dbuf[i] = jnp.zeros((16,), jnp.float32)
        return 0
      lax.fori_loop(0, 1024, zb, 0)
      for t in range(6):
        pltpu.sync_copy(dbuf, acc.at[pl.ds(sub * STRIPE + t * 1024, 1024), :])
      pltpu.sync_copy(dbuf.at[pl.ds(0, 128)],
                      acc.at[pl.ds(sub * STRIPE + 6144, 128), :])
      plsc.subcore_barrier()

      if split:
        ch = 0
        m2 = m_tot // 2
        mb0 = core * m2 + sub
        cnt = (m2 - sub + NS - 1) // NS
      else:
        ch = core * cpc + cc
        mb0 = sub
        cnt = (m_tot - sub + NS - 1) // NS
      col = 16 * ch

      def mbody(m, _):
        mb = mb0 + m * NS
        base = mb * 1024
        row = mb * 8
        if full:
          pltpu.sync_copy(data_h.at[pl.ds(base, 1024), :], dbuf)
        else:
          pltpu.sync_copy(data_h.at[pl.ds(base, 1024), pl.ds(col, 16)], dbuf)
        pltpu.sync_copy(idx_h.at[0, pl.ds(row, 8), :], ib0)
        pltpu.sync_copy(idx_h.at[1, pl.ds(row, 8), :], ib1)
        pltpu.sync_copy(idx_h.at[2, pl.ds(row, 8), :], ib2)
        for ib in (ib0, ib1, ib2):
          for r in range(8):
            pltpu.sync_copy(dbuf.at[pl.ds(128 * r, 128)],
                            acc.at[ib.at[r]], add=True)
        return 0
      lax.fori_loop(0, cnt, mbody, 0)
      plsc.subcore_barrier()

      oi = core if split else ch
      pltpu.sync_copy(acc.at[pl.ds(sub * STRIPE, STRIPE), :],
                      out_h.at[oi, pl.ds(sub * STRIPE, STRIPE), :])
      plsc.subcore_barrier()

  return pl.kernel(
      body,
      out_type=jax.ShapeDtypeStruct((n_out, VPAD, 16), jnp.float32),
      mesh=_mesh(),
      scratch_types=[
          pltpu.VMEM_SHARED((VPAD, 16), jnp.float32),
          pltpu.VMEM((1024, 16), jnp.float32),
          pltpu.VMEM((8, 128), jnp.int32),
          pltpu.VMEM((8, 128), jnp.int32),
          pltpu.VMEM((8, 128), jnp.int32),
      ],
      compiler_params=_SC_PARAMS,
  )(data, idx3)


_HI = lax.Precision.HIGHEST


def _sel(entries):
  import numpy as np
  m = np.zeros((128, 128), np.float32)
  for r, c in entries:
    m[r, c] = 1.0
  return m


_C1 = _sel([(16 * s + (k + 1) % 3, 16 * s + k)
            for s in range(8) for k in range(3)])
_C2 = _sel([(16 * s + (k + 2) % 3, 16 * s + k)
            for s in range(8) for k in range(3)])
_S6 = _sel([(16 * s + k, 16 * s + j)
            for s in range(8) for k in range(3) for j in range(7)])
_P35 = _sel([(16 * s + k, 16 * s + 3 + k)
             for s in range(8) for k in range(3)])


def _bcast_mat(cl, dout):
  import numpy as np
  m = np.zeros((128, 8 * dout), np.float32)
  for q in range(8):
    for j in range(dout):
      m[16 * q + cl, dout * q + j] = 1.0
  return m


_BRF = 256
_BRV = 448
_NBV = 28


def _ceil(a, b):
  return (a + b - 1) // b


def _tc_features(gpos_p):
  nb = _ceil(F_REAL[0] // 8, _BRF)

  def body(g0r, g1r, g2r, c1r, c2r, s6r, p35r, o_r):
    v0, v1, v2 = g0r[0], g1r[0], g2r[0]
    e1 = v1 - v0
    e2 = v2 - v0
    a1 = jnp.dot(e1, c1r[...], precision=_HI)
    a2 = jnp.dot(e1, c2r[...], precision=_HI)
    b1_ = jnp.dot(e2, c1r[...], precision=_HI)
    b2_ = jnp.dot(e2, c2r[...], precision=_HI)
    n = a1 * b2_ - a2 * b1_
    nrm = jnp.sqrt(jnp.dot(n * n, s6r[...], precision=_HI))
    inv = 1.0 / (nrm + 1e-8)
    nshift = jnp.dot(n * inv, p35r[...], precision=_HI)
    cent = (v0 + v1 + v2) * (1.0 / 3.0)
    lane = lax.broadcasted_iota(jnp.int32, (_BRF, 128), 1)
    lanem = lane % 16
    fid = ((pl.program_id(0) * _BRF
            + lax.broadcasted_iota(jnp.int32, (_BRF, 128), 0)) * 8
           + lane // 16)
    res = jnp.where(lanem < 3, cent, 0.0) + nshift
    res = res + jnp.where(lanem == 6, 0.5 * nrm, 0.0)
    res = res + jnp.where(lanem == 7, 1.0, 0.0)
    for li, thr in ((8, F_REAL[1]), (9, F_REAL[2]), (10, F_REAL[3])):
      res = res + jnp.where((lanem == li) & (fid < thr), 1.0, 0.0)
    o_r[...] = res

  gspec = lambda j: pl.BlockSpec((1, _BRF, 128), lambda i, j=j: (j, i, 0))
  cspec = pl.BlockSpec((128, 128), lambda i: (0, 0))
  return pl.pallas_call(
      body,
      grid=(nb,),
      in_specs=[gspec(0), gspec(1), gspec(2), cspec, cspec, cspec, cspec],
      out_specs=pl.BlockSpec((_BRF, 128), lambda i: (i, 0)),
      out_shape=jax.ShapeDtypeStruct((F_PAD[0] // 8, 128), jnp.float32),
  )(gpos_p, gpos_p, gpos_p, jnp.asarray(_C1), jnp.asarray(_C2),
    jnp.asarray(_S6), jnp.asarray(_P35))


def _tc_vertex1(sums1_p, b7):
  def body(sa_r, sb_r, b7r, o_r):
    s = sa_r[0] + sb_r[0]
    ivb = 1.0 / jnp.maximum(jnp.dot(s, b7r[...], precision=_HI), 1.0)
    o_r[...] = s * ivb

  pspec = lambda p: pl.BlockSpec((1, _BRV, 128), lambda i, p=p: (p, i, 0))
  return pl.pallas_call(
      body,
      grid=(_NBV,),
      in_specs=[pspec(0), pspec(1), pl.BlockSpec((128, 128), lambda i: (0, 0))],
      out_specs=pl.BlockSpec((_BRV, 128), lambda i: (i, 0)),
      out_shape=jax.ShapeDtypeStruct((VPAD // 8, 128), jnp.float32),
  )(sums1_p, sums1_p, b7)


def _tc_vertex(sums_p, sums1_p, wstack, bl, dout):
  lo = 8 * dout

  def body(s_r, w_r, sa_r, sb_r, bl_r, o_r):
    ci = pl.program_id(1)
    p = jnp.dot(s_r[0], w_r[0], preferred_element_type=jnp.float32)
    acc = jnp.where(ci == 0, p, o_r[...] + p)
    s1 = sa_r[0] + sb_r[0]
    ivb = 1.0 / jnp.maximum(jnp.dot(s1, bl_r[...], precision=_HI), 1.0)
    o_r[...] = jnp.where(ci == 3, acc * ivb, acc)

  p1spec = lambda p: pl.BlockSpec((1, _BRV, 128), lambda i, c, p=p: (p, i, 0))
  return pl.pallas_call(
      body,
      grid=(_NBV, 4),
      in_specs=[
          pl.BlockSpec((1, _BRV, 128), lambda i, c: (c, i, 0)),
          pl.BlockSpec((1, 128, lo), lambda i, c: (c, 0, 0)),
          p1spec(0), p1spec(1),
          pl.BlockSpec((128, lo), lambda i, c: (0, 0)),
      ],
      out_specs=pl.BlockSpec((_BRV, lo), lambda i, c: (i, 0)),
      out_shape=jax.ShapeDtypeStruct((VPAD // 8, lo), jnp.float32),
  )(sums_p, wstack, sums1_p, sums1_p, bl)


def _tc_face(feats_p, g_p, bdws, bt, rows, lf, lg, lo, rows_out,
             bdwn=None, relu=True):
  nb = _ceil(rows, _BRF)

  def body(fl, g0l, g1l, g2l, w_r, *rest):
    if bdwn is not None:
      wn_r = rest[0]
      rest = rest[1:]
    b_r, o_r = rest
    agg = (g0l[0] + g1l[0] + g2l[0]) * (1.0 / 3.0)
    z = jnp.dot(fl[...], w_r[...], preferred_element_type=jnp.float32)
    if bdwn is not None:
      z = z + jnp.dot(agg, wn_r[...], preferred_element_type=jnp.float32)
    else:
      z = z + agg
    z = z + b_r[...]
    o_r[...] = jnp.maximum(z, 0.0) if relu else z

  in_specs = [pl.BlockSpec((_BRF, lf), lambda i: (i, 0))]
  args = [feats_p]
  in_specs += [pl.BlockSpec((1, _BRF, lg), lambda i, j=j: (j, i, 0))
               for j in range(3)]
  args += [g_p, g_p, g_p]
  in_specs.append(pl.BlockSpec((lf, lo), lambda i: (0, 0)))
  args.append(bdws)
  if bdwn is not None:
    in_specs.append(pl.BlockSpec((lg, lo), lambda i: (0, 0)))
    args.append(bdwn)
  in_specs.append(pl.BlockSpec((1, lo), lambda i: (0, 0)))
  args.append(bt.reshape(1, lo))

  return pl.pallas_call(
      body,
      grid=(nb,),
      in_specs=in_specs,
      out_specs=pl.BlockSpec((_BRF, lo), lambda i: (i, 0)),
      out_shape=jax.ShapeDtypeStruct((rows_out, lo), jnp.float32),
  )(*args)


def _tc_pool(zlo, zhi, rows_out):
  nb = _ceil(zlo.shape[0], _BRF)

  def body(a_r, b_r, o_r):
    o_r[...] = 0.5 * (a_r[...] + b_r[...])

  spec = pl.BlockSpec((_BRF, 128), lambda i: (i, 0))
  return pl.pallas_call(
      body,
      grid=(nb,),
      in_specs=[spec, spec],
      out_specs=spec,
      out_shape=jax.ShapeDtypeStruct((rows_out, 128), jnp.float32),
  )(zlo, zhi)


def _kron8(w):
  return jnp.kron(jnp.eye(8, dtype=jnp.float32), w)


def _kron2(w):
  return jnp.kron(jnp.eye(2, dtype=jnp.float32), w)


def kernel(pos, faces, W1s, W1n, b1, W2s, W2n, b2, W3s, W3n, b3,
           W4s, W4n, b4):
  facesT = faces.T.astype(jnp.int32)
  idx = []
  for l in range(4):
    a = jnp.concatenate(
        [facesT[:, :F_REAL[l]],
         jnp.full((3, F_PAD[l] - F_REAL[l]), DUMMY, jnp.int32)], axis=1)
    idx.append(a.reshape(3, F_PAD[l] // 128, 128))

  pos16 = jnp.pad(pos.astype(jnp.float32), ((0, VPAD - V), (0, 13)))
  w1s16 = jnp.pad(W1s, ((0, 9), (0, 0)))
  w1n16 = jnp.pad(W1n, ((0, 9), (0, 0)))
  b7 = jnp.asarray(_bcast_mat(7, 16))
  bl2 = jnp.asarray(_bcast_mat(8, 64))
  bl3 = jnp.asarray(_bcast_mat(9, 64))
  bl4 = jnp.asarray(_bcast_mat(10, 16))
  wn2 = jnp.stack([_kron8(W2n[16 * c:16 * c + 16]) for c in range(4)])
  wn3 = jnp.stack([_kron8(W3n[16 * c:16 * c + 16]) for c in range(4)])
  wn4 = jnp.stack([_kron8(W4n[16 * c:16 * c + 16]) for c in range(4)])

  gpos = _sc_gather(pos16, idx[0], F_PAD[0], 16)
  feats1p = _tc_features(gpos.reshape(3, F_PAD[0] // 8, 128))
  sums1 = _sc_scatter(feats1p.reshape(F_PAD[0], 16), idx[0], F_PAD[0], 1,
                      split=True)
  sums1p = sums1.reshape(2, VPAD // 8, 128)
  vt1p = _tc_vertex1(sums1p, b7)
  g1 = _sc_gather(vt1p.reshape(VPAD, 16), idx[0], F_PAD[0], 16)
  z1 = _tc_face(feats1p, g1.reshape(3, F_PAD[0] // 8, 128),
                _kron8(w1s16), jnp.tile(b1, 8), F_REAL[0] // 8,
                128, 128, 512, F_PAD[0] // 8, bdwn=_kron8(w1n16), relu=True)
  z1p2 = z1.reshape(F_PAD[0] // 2, 128)
  feats2p = _tc_pool(z1p2[:F_REAL[0] // 4],
                     z1p2[F_REAL[0] // 4:F_REAL[0] // 2], F_PAD[1] // 2)
  feats2 = feats2p.reshape(F_PAD[1], 64)

  sums2 = _sc_scatter(feats2, idx[1], F_PAD[1], 4, split=False)
  vt2p = _tc_vertex(sums2.reshape(4, VPAD // 8, 128), sums1p, wn2, bl2, 64)
  g2 = _sc_gather(vt2p.reshape(VPAD, 64), idx[1], F_PAD[1], 64)
  z2 = _tc_face(feats2p, g2.reshape(3, F_PAD[1] // 2, 128),
                _kron2(W2s), jnp.tile(b2, 2), F_REAL[1] // 2,
                128, 128, 128, F_PAD[1] // 2, relu=True)
  feats3p = _tc_pool(z2[:F_REAL[1] // 4],
                     z2[F_REAL[1] // 4:F_REAL[1] // 2], F_PAD[2] // 2)
  feats3 = feats3p.reshape(F_PAD[2], 64)

  sums3 = _sc_scatter(feats3, idx[2], F_PAD[2], 4, split=False)
  vt3p = _tc_vertex(sums3.reshape(4, VPAD // 8, 128), sums1p, wn3, bl3, 64)
  g3 = _sc_gather(vt3p.reshape(VPAD, 64), idx[2], F_PAD[2], 64)
  z3 = _tc_face(feats3p, g3.reshape(3, F_PAD[2] // 2, 128),
                _kron2(W3s), jnp.tile(b3, 2), F_REAL[2] // 2,
                128, 128, 128, F_PAD[2] // 2, relu=True)
  feats4p = _tc_pool(z3[:F_REAL[2] // 4],
                     z3[F_REAL[2] // 4:F_REAL[2] // 2], F_PAD[3] // 2)
  feats4 = feats4p.reshape(F_PAD[3], 64)

  sums4 = _sc_scatter(feats4, idx[3], F_PAD[3], 4, split=False)
  vt4p = _tc_vertex(sums4.reshape(4, VPAD // 8, 128), sums1p, wn4, bl4, 16)
  g4 = _sc_gather(vt4p.reshape(VPAD, 16), idx[3], F_PAD[3], 16)
  outp = _tc_face(feats4.reshape(F_PAD[3] // 8, 512),
                  g4.reshape(3, F_PAD[3] // 8, 128),
                  _kron8(W4s), jnp.tile(b4, 8), F_REAL[3] // 8,
                  512, 128, 128, F_REAL[3] // 8, relu=False)

  return (faces[:F_REAL[3]], outp.reshape(F_REAL[3], 16))


# speedup vs baseline: 8.8316x; 1.3081x over previous
import functools

import jax
import jax.numpy as jnp
from jax import lax
from jax.experimental import pallas as pl
from jax.experimental.pallas import tpu as pltpu
from jax.experimental.pallas import tpu_sc as plsc

V = 100000
VPAD = 100352
DUMMY = 100000
STRIPE = VPAD // 16
F_REAL = (200000, 100000, 50000, 25000)
F_PAD = (200704, 100352, 50176, 25600)
NC, NS = 2, 16
NW = NC * NS
BF = 1000
BV = 512

_SC_PARAMS = pltpu.CompilerParams(use_tc_tiling_on_sc=False)

_mesh = functools.partial(
    plsc.VectorSubcoreMesh,
    core_axis_name="c", subcore_axis_name="s", num_cores=NC, num_subcores=NS)


def _sc_gather(table, idx3, fp, d):
  m_tot = fp // 1024

  def body(table_h, idx_h, out_h, ibuf, rows, sem):
    core = lax.axis_index("c")
    sub = lax.axis_index("s")
    wid = sub * NC + core
    cnt = (m_tot - wid + NW - 1) // NW
    for j in range(3):
      def mbody(m, _):
        mb = wid + m * NW
        base = mb * 1024
        row = mb * 8
        pltpu.sync_copy(idx_h.at[j, pl.ds(row, 8), :], ibuf)
        cps = [pltpu.async_copy(table_h.at[ibuf.at[r]],
                                rows.at[pl.ds(128 * r, 128)], sem)
               for r in range(8)]
        for cp in cps:
          cp.wait()
        pltpu.sync_copy(rows, out_h.at[j, pl.ds(base, 1024), :])
        return 0
      lax.fori_loop(0, cnt, mbody, 0)

  return pl.kernel(
      body,
      out_type=jax.ShapeDtypeStruct((3, fp, d), jnp.float32),
      mesh=_mesh(),
      scratch_types=[
          pltpu.VMEM((8, 128), jnp.int32),
          pltpu.VMEM((1024, d), jnp.float32),
          pltpu.SemaphoreType.DMA,
      ],
      compiler_params=_SC_PARAMS,
  )(table, idx3)


def _sc_scatter(data, idx3, fp, nchunks, split):
  n_out = 2 if split else nchunks
  cpc = 1 if split else nchunks // 2
  m_tot = fp // 1024
  full = split

  def body(data_h, idx_h, out_h, acc, dbuf, ib0, ib1, ib2):
    core = lax.axis_index("c")
    sub = lax.axis_index("s")

    for cc in range(cpc):
      def zb(i, _):
        dbuf[i] = jnp.zeros((16,), jnp.float32)
        return 0
      lax.fori_loop(0, 1024, zb, 0)
      for t in range(6):
        pltpu.sync_copy(dbuf, acc.at[pl.ds(sub * STRIPE + t * 1024, 1024), :])
      pltpu.sync_copy(dbuf.at[pl.ds(0, 128)],
                      acc.at[pl.ds(sub * STRIPE + 6144, 128), :])
      plsc.subcore_barrier()

      if split:
        ch = 0
        m2 = m_tot // 2
        mb0 = core * m2 + sub
        cnt = (m2 - sub + NS - 1) // NS
      else:
        ch = core * cpc + cc
        mb0 = sub
        cnt = (m_tot - sub + NS - 1) // NS
      col = 16 * ch

      def mbody(m, _):
        mb = mb0 + m * NS
        base = mb * 1024
        row = mb * 8
        if full:
          pltpu.sync_copy(data_h.at[pl.ds(base, 1024), :], dbuf)
        else:
          pltpu.sync_copy(data_h.at[pl.ds(base, 1024), pl.ds(col, 16)], dbuf)
        pltpu.sync_copy(idx_h.at[0, pl.ds(row, 8), :], ib0)
        pltpu.sync_copy(idx_h.at[1, pl.ds(row, 8), :], ib1)
        pltpu.sync_copy(idx_h.at[2, pl.ds(row, 8), :], ib2)
        for ib in (ib0, ib1, ib2):
          for r in range(8):
            pltpu.sync_copy(dbuf.at[pl.ds(128 * r, 128)],
                            acc.at[ib.at[r]], add=True)
        return 0
      lax.fori_loop(0, cnt, mbody, 0)
      plsc.subcore_barrier()

      oi = core if split else ch
      pltpu.sync_copy(acc.at[pl.ds(sub * STRIPE, STRIPE), :],
                      out_h.at[oi, pl.ds(sub * STRIPE, STRIPE), :])
      plsc.subcore_barrier()

  return pl.kernel(
      body,
      out_type=jax.ShapeDtypeStruct((n_out, VPAD, 16), jnp.float32),
      mesh=_mesh(),
      scratch_types=[
          pltpu.VMEM_SHARED((VPAD, 16), jnp.float32),
          pltpu.VMEM((1024, 16), jnp.float32),
          pltpu.VMEM((8, 128), jnp.int32),
          pltpu.VMEM((8, 128), jnp.int32),
          pltpu.VMEM((8, 128), jnp.int32),
      ],
      compiler_params=_SC_PARAMS,
  )(data, idx3)


_HI = lax.Precision.HIGHEST


def _sel(entries):
  import numpy as np
  m = np.zeros((128, 128), np.float32)
  for r, c in entries:
    m[r, c] = 1.0
  return m


_C1 = _sel([(16 * s + (k + 1) % 3, 16 * s + k)
            for s in range(8) for k in range(3)])
_C2 = _sel([(16 * s + (k + 2) % 3, 16 * s + k)
            for s in range(8) for k in range(3)])
_S6 = _sel([(16 * s + k, 16 * s + j)
            for s in range(8) for k in range(3) for j in range(7)])
_P35 = _sel([(16 * s + k, 16 * s + 3 + k)
             for s in range(8) for k in range(3)])


def _bcast_mat(cl, dout):
  import numpy as np
  m = np.zeros((128, 8 * dout), np.float32)
  for q in range(8):
    for j in range(dout):
      m[16 * q + cl, dout * q + j] = 1.0
  return m


_BRF = 512
_BRV = 448
_NBV = 28


def _ceil(a, b):
  return (a + b - 1) // b


def _tc_features(gpos_p):
  nb = _ceil(F_REAL[0] // 8, _BRF)

  def body(g0r, g1r, g2r, c1r, c2r, s6r, p35r, o_r):
    v0, v1, v2 = g0r[0], g1r[0], g2r[0]
    e1 = v1 - v0
    e2 = v2 - v0
    a1 = jnp.dot(e1, c1r[...], precision=_HI)
    a2 = jnp.dot(e1, c2r[...], precision=_HI)
    b1_ = jnp.dot(e2, c1r[...], precision=_HI)
    b2_ = jnp.dot(e2, c2r[...], precision=_HI)
    n = a1 * b2_ - a2 * b1_
    nrm = jnp.sqrt(jnp.dot(n * n, s6r[...], precision=_HI))
    inv = 1.0 / (nrm + 1e-8)
    nshift = jnp.dot(n * inv, p35r[...], precision=_HI)
    cent = (v0 + v1 + v2) * (1.0 / 3.0)
    lane = lax.broadcasted_iota(jnp.int32, (_BRF, 128), 1)
    lanem = lane % 16
    fid = ((pl.program_id(0) * _BRF
            + lax.broadcasted_iota(jnp.int32, (_BRF, 128), 0)) * 8
           + lane // 16)
    res = jnp.where(lanem < 3, cent, 0.0) + nshift
    res = res + jnp.where(lanem == 6, 0.5 * nrm, 0.0)
    res = res + jnp.where(lanem == 7, 1.0, 0.0)
    for li, thr in ((8, F_REAL[1]), (9, F_REAL[2]), (10, F_REAL[3])):
      res = res + jnp.where((lanem == li) & (fid < thr), 1.0, 0.0)
    o_r[...] = res

  gspec = lambda j: pl.BlockSpec((1, _BRF, 128), lambda i, j=j: (j, i, 0))
  cspec = pl.BlockSpec((128, 128), lambda i: (0, 0))
  return pl.pallas_call(
      body,
      grid=(nb,),
      in_specs=[gspec(0), gspec(1), gspec(2), cspec, cspec, cspec, cspec],
      out_specs=pl.BlockSpec((_BRF, 128), lambda i: (i, 0)),
      out_shape=jax.ShapeDtypeStruct((F_PAD[0] // 8, 128), jnp.float32),
  )(gpos_p, gpos_p, gpos_p, jnp.asarray(_C1), jnp.asarray(_C2),
    jnp.asarray(_S6), jnp.asarray(_P35))


def _tc_vertex1(sums1_p, b7):
  def body(sa_r, sb_r, b7r, o_r):
    s = sa_r[0] + sb_r[0]
    ivb = 1.0 / jnp.maximum(jnp.dot(s, b7r[...]), 1.0)
    o_r[...] = s * ivb

  pspec = lambda p: pl.BlockSpec((1, _BRV, 128), lambda i, p=p: (p, i, 0))
  return pl.pallas_call(
      body,
      grid=(_NBV,),
      in_specs=[pspec(0), pspec(1), pl.BlockSpec((128, 128), lambda i: (0, 0))],
      out_specs=pl.BlockSpec((_BRV, 128), lambda i: (i, 0)),
      out_shape=jax.ShapeDtypeStruct((VPAD // 8, 128), jnp.float32),
  )(sums1_p, sums1_p, b7)


def _tc_vertex(sums_p, sums1_p, wstack, bl, dout):
  lo = 8 * dout

  def body(s_r, w_r, sa_r, sb_r, bl_r, o_r):
    ci = pl.program_id(1)
    p = jnp.dot(s_r[0], w_r[0], preferred_element_type=jnp.float32)
    acc = jnp.where(ci == 0, p, o_r[...] + p)
    s1 = sa_r[0] + sb_r[0]
    ivb = 1.0 / jnp.maximum(jnp.dot(s1, bl_r[...]), 1.0)
    o_r[...] = jnp.where(ci == 3, acc * ivb, acc)

  p1spec = lambda p: pl.BlockSpec((1, _BRV, 128), lambda i, c, p=p: (p, i, 0))
  return pl.pallas_call(
      body,
      grid=(_NBV, 4),
      in_specs=[
          pl.BlockSpec((1, _BRV, 128), lambda i, c: (c, i, 0)),
          pl.BlockSpec((1, 128, lo), lambda i, c: (c, 0, 0)),
          p1spec(0), p1spec(1),
          pl.BlockSpec((128, lo), lambda i, c: (0, 0)),
      ],
      out_specs=pl.BlockSpec((_BRV, lo), lambda i, c: (i, 0)),
      out_shape=jax.ShapeDtypeStruct((VPAD // 8, lo), jnp.float32),
  )(sums_p, wstack, sums1_p, sums1_p, bl)


def _tc_face(feats_p, g_p, bdws, bt, rows, lf, lg, lo, rows_out,
             bdwn=None, relu=True):
  nb = _ceil(rows, _BRF)

  def body(fl, g0l, g1l, g2l, w_r, *rest):
    if bdwn is not None:
      wn_r = rest[0]
      rest = rest[1:]
    b_r, o_r = rest
    agg = (g0l[0] + g1l[0] + g2l[0]) * (1.0 / 3.0)
    z = jnp.dot(fl[...], w_r[...], preferred_element_type=jnp.float32)
    if bdwn is not None:
      z = z + jnp.dot(agg, wn_r[...], preferred_element_type=jnp.float32)
    else:
      z = z + agg
    z = z + b_r[...]
    o_r[...] = jnp.maximum(z, 0.0) if relu else z

  in_specs = [pl.BlockSpec((_BRF, lf), lambda i: (i, 0))]
  args = [feats_p]
  in_specs += [pl.BlockSpec((1, _BRF, lg), lambda i, j=j: (j, i, 0))
               for j in range(3)]
  args += [g_p, g_p, g_p]
  in_specs.append(pl.BlockSpec((lf, lo), lambda i: (0, 0)))
  args.append(bdws)
  if bdwn is not None:
    in_specs.append(pl.BlockSpec((lg, lo), lambda i: (0, 0)))
    args.append(bdwn)
  in_specs.append(pl.BlockSpec((1, lo), lambda i: (0, 0)))
  args.append(bt.reshape(1, lo))

  return pl.pallas_call(
      body,
      grid=(nb,),
      in_specs=in_specs,
      out_specs=pl.BlockSpec((_BRF, lo), lambda i: (i, 0)),
      out_shape=jax.ShapeDtypeStruct((rows_out, lo), jnp.float32),
  )(*args)


def _tc_pool(zlo, zhi, rows_out):
  nb = _ceil(zlo.shape[0], _BRF)

  def body(a_r, b_r, o_r):
    o_r[...] = 0.5 * (a_r[...] + b_r[...])

  spec = pl.BlockSpec((_BRF, 128), lambda i: (i, 0))
  return pl.pallas_call(
      body,
      grid=(nb,),
      in_specs=[spec, spec],
      out_specs=spec,
      out_shape=jax.ShapeDtypeStruct((rows_out, 128), jnp.float32),
  )(zlo, zhi)


def _kron8(w):
  return jnp.kron(jnp.eye(8, dtype=jnp.float32), w)


def _kron2(w):
  return jnp.kron(jnp.eye(2, dtype=jnp.float32), w)


def kernel(pos, faces, W1s, W1n, b1, W2s, W2n, b2, W3s, W3n, b3,
           W4s, W4n, b4):
  facesT = faces.T.astype(jnp.int32)
  idx = []
  for l in range(4):
    a = jnp.concatenate(
        [facesT[:, :F_REAL[l]],
         jnp.full((3, F_PAD[l] - F_REAL[l]), DUMMY, jnp.int32)], axis=1)
    idx.append(a.reshape(3, F_PAD[l] // 128, 128))

  pos16 = jnp.pad(pos.astype(jnp.float32), ((0, VPAD - V), (0, 13)))
  w1s16 = jnp.pad(W1s, ((0, 9), (0, 0)))
  w1n16 = jnp.pad(W1n, ((0, 9), (0, 0)))
  b7 = jnp.asarray(_bcast_mat(7, 16))
  bl2 = jnp.asarray(_bcast_mat(8, 64))
  bl3 = jnp.asarray(_bcast_mat(9, 64))
  bl4 = jnp.asarray(_bcast_mat(10, 16))
  wn2 = jnp.stack([_kron8(W2n[16 * c:16 * c + 16]) for c in range(4)])
  wn3 = jnp.stack([_kron8(W3n[16 * c:16 * c + 16]) for c in range(4)])
  wn4 = jnp.stack([_kron8(W4n[16 * c:16 * c + 16]) for c in range(4)])

  gpos = _sc_gather(pos16, idx[0], F_PAD[0], 16)
  feats1p = _tc_features(gpos.reshape(3, F_PAD[0] // 8, 128))
  sums1 = _sc_scatter(feats1p.reshape(F_PAD[0], 16), idx[0], F_PAD[0], 1,
                      split=True)
  sums1p = sums1.reshape(2, VPAD // 8, 128)
  vt1p = _tc_vertex1(sums1p, b7)
  g1 = _sc_gather(vt1p.reshape(VPAD, 16), idx[0], F_PAD[0], 16)
  z1 = _tc_face(feats1p, g1.reshape(3, F_PAD[0] // 8, 128),
                _kron8(w1s16), jnp.tile(b1, 8), F_REAL[0] // 8,
                128, 128, 512, F_PAD[0] // 8, bdwn=_kron8(w1n16), relu=True)
  z1p2 = z1.reshape(F_PAD[0] // 2, 128)
  feats2p = _tc_pool(z1p2[:F_REAL[0] // 4],
                     z1p2[F_REAL[0] // 4:F_REAL[0] // 2], F_PAD[1] // 2)
  feats2 = feats2p.reshape(F_PAD[1], 64)

  sums2 = _sc_scatter(feats2, idx[1], F_PAD[1], 4, split=False)
  vt2p = _tc_vertex(sums2.reshape(4, VPAD // 8, 128), sums1p, wn2, bl2, 64)
  g2 = _sc_gather(vt2p.reshape(VPAD, 64), idx[1], F_PAD[1], 64)
  z2 = _tc_face(feats2p, g2.reshape(3, F_PAD[1] // 2, 128),
                _kron2(W2s), jnp.tile(b2, 2), F_REAL[1] // 2,
                128, 128, 128, F_PAD[1] // 2, relu=True)
  feats3p = _tc_pool(z2[:F_REAL[1] // 4],
                     z2[F_REAL[1] // 4:F_REAL[1] // 2], F_PAD[2] // 2)
  feats3 = feats3p.reshape(F_PAD[2], 64)

  sums3 = _sc_scatter(feats3, idx[2], F_PAD[2], 4, split=False)
  vt3p = _tc_vertex(sums3.reshape(4, VPAD // 8, 128), sums1p, wn3, bl3, 64)
  g3 = _sc_gather(vt3p.reshape(VPAD, 64), idx[2], F_PAD[2], 64)
  z3 = _tc_face(feats3p, g3.reshape(3, F_PAD[2] // 2, 128),
                _kron2(W3s), jnp.tile(b3, 2), F_REAL[2] // 2,
                128, 128, 128, F_PAD[2] // 2, relu=True)
  feats4p = _tc_pool(z3[:F_REAL[2] // 4],
                     z3[F_REAL[2] // 4:F_REAL[2] // 2], F_PAD[3] // 2)
  feats4 = feats4p.reshape(F_PAD[3], 64)

  sums4 = _sc_scatter(feats4, idx[3], F_PAD[3], 4, split=False)
  vt4p = _tc_vertex(sums4.reshape(4, VPAD // 8, 128), sums1p, wn4, bl4, 16)
  g4 = _sc_gather(vt4p.reshape(VPAD, 16), idx[3], F_PAD[3], 16)
  outp = _tc_face(feats4.reshape(F_PAD[3] // 8, 512),
                  g4.reshape(3, F_PAD[3] // 8, 128),
                  _kron8(W4s), jnp.tile(b4, 8), F_REAL[3] // 8,
                  512, 128, 128, F_REAL[3] // 8, relu=False)

  return (faces[:F_REAL[3]], outp.reshape(F_REAL[3], 16))


# speedup vs baseline: 9.0578x; 1.0256x over previous
import functools

import jax
import jax.numpy as jnp
from jax import lax
from jax.experimental import pallas as pl
from jax.experimental.pallas import tpu as pltpu
from jax.experimental.pallas import tpu_sc as plsc

V = 100000
VPAD = 100352
DUMMY = 100000
STRIPE = VPAD // 16
F_REAL = (200000, 100000, 50000, 25000)
F_PAD = (200704, 100352, 50176, 25600)
NC, NS = 2, 16
NW = NC * NS
BF = 1000
BV = 512

_SC_PARAMS = pltpu.CompilerParams(use_tc_tiling_on_sc=False)

_mesh = functools.partial(
    plsc.VectorSubcoreMesh,
    core_axis_name="c", subcore_axis_name="s", num_cores=NC, num_subcores=NS)


def _sc_gather(table, idx3, fp, d):
  m_tot = fp // 1024

  def body(table_h, idx_h, out_h, ibuf, rows, sem):
    core = lax.axis_index("c")
    sub = lax.axis_index("s")
    wid = sub * NC + core
    cnt = (m_tot - wid + NW - 1) // NW
    for j in range(3):
      def mbody(m, _):
        mb = wid + m * NW
        base = mb * 1024
        row = mb * 8
        pltpu.sync_copy(idx_h.at[j, pl.ds(row, 8), :], ibuf)
        cps = [pltpu.async_copy(table_h.at[ibuf.at[r]],
                                rows.at[pl.ds(128 * r, 128)], sem)
               for r in range(8)]
        for cp in cps:
          cp.wait()
        pltpu.sync_copy(rows, out_h.at[j, pl.ds(base, 1024), :])
        return 0
      lax.fori_loop(0, cnt, mbody, 0)

  return pl.kernel(
      body,
      out_type=jax.ShapeDtypeStruct((3, fp, d), jnp.float32),
      mesh=_mesh(),
      scratch_types=[
          pltpu.VMEM((8, 128), jnp.int32),
          pltpu.VMEM((1024, d), jnp.float32),
          pltpu.SemaphoreType.DMA,
      ],
      compiler_params=_SC_PARAMS,
  )(table, idx3)


def _sc_scatter(data, idx3, fp, nchunks, split):
  n_out = 2 if split else nchunks
  cpc = 1 if split else nchunks // 2
  m_tot = fp // 1024
  full = split

  def body(data_h, idx_h, out_h, acc, dbuf, ib0, ib1, ib2, sem):
    core = lax.axis_index("c")
    sub = lax.axis_index("s")

    for cc in range(cpc):
      def zb(i, _):
        dbuf[i] = jnp.zeros((16,), jnp.float32)
        return 0
      lax.fori_loop(0, 1024, zb, 0)
      for t in range(6):
        pltpu.sync_copy(dbuf, acc.at[pl.ds(sub * STRIPE + t * 1024, 1024), :])
      pltpu.sync_copy(dbuf.at[pl.ds(0, 128)],
                      acc.at[pl.ds(sub * STRIPE + 6144, 128), :])
      plsc.subcore_barrier()

      if split:
        ch = 0
        m2 = m_tot // 2
        mb0 = core * m2 + sub
        cnt = (m2 - sub + NS - 1) // NS
      else:
        ch = core * cpc + cc
        mb0 = sub
        cnt = (m_tot - sub + NS - 1) // NS
      col = 16 * ch

      def mbody(m, _):
        mb = mb0 + m * NS
        base = mb * 1024
        row = mb * 8
        if full:
          pltpu.sync_copy(data_h.at[pl.ds(base, 1024), :], dbuf)
        else:
          pltpu.sync_copy(data_h.at[pl.ds(base, 1024), pl.ds(col, 16)], dbuf)
        pltpu.sync_copy(idx_h.at[0, pl.ds(row, 8), :], ib0)
        pltpu.sync_copy(idx_h.at[1, pl.ds(row, 8), :], ib1)
        pltpu.sync_copy(idx_h.at[2, pl.ds(row, 8), :], ib2)
        cps = [pltpu.async_copy(dbuf.at[pl.ds(128 * r, 128)],
                                acc.at[ib.at[r]], sem, add=True)
               for ib in (ib0, ib1, ib2) for r in range(8)]
        for cp in cps:
          cp.wait()
        return 0
      lax.fori_loop(0, cnt, mbody, 0)
      plsc.subcore_barrier()

      oi = core if split else ch
      pltpu.sync_copy(acc.at[pl.ds(sub * STRIPE, STRIPE), :],
                      out_h.at[oi, pl.ds(sub * STRIPE, STRIPE), :])
      plsc.subcore_barrier()

  return pl.kernel(
      body,
      out_type=jax.ShapeDtypeStruct((n_out, VPAD, 16), jnp.float32),
      mesh=_mesh(),
      scratch_types=[
          pltpu.VMEM_SHARED((VPAD, 16), jnp.float32),
          pltpu.VMEM((1024, 16), jnp.float32),
          pltpu.VMEM((8, 128), jnp.int32),
          pltpu.VMEM((8, 128), jnp.int32),
          pltpu.VMEM((8, 128), jnp.int32),
          pltpu.SemaphoreType.DMA,
      ],
      compiler_params=_SC_PARAMS,
  )(data, idx3)


_HI = lax.Precision.HIGHEST


def _sel(entries):
  import numpy as np
  m = np.zeros((128, 128), np.float32)
  for r, c in entries:
    m[r, c] = 1.0
  return m


_C1 = _sel([(16 * s + (k + 1) % 3, 16 * s + k)
            for s in range(8) for k in range(3)])
_C2 = _sel([(16 * s + (k + 2) % 3, 16 * s + k)
            for s in range(8) for k in range(3)])
_S6 = _sel([(16 * s + k, 16 * s + j)
            for s in range(8) for k in range(3) for j in range(7)])
_P35 = _sel([(16 * s + k, 16 * s + 3 + k)
             for s in range(8) for k in range(3)])


def _bcast_mat(cl, dout):
  import numpy as np
  m = np.zeros((128, 8 * dout), np.float32)
  for q in range(8):
    for j in range(dout):
      m[16 * q + cl, dout * q + j] = 1.0
  return m


_BRF = 512
_BRV = 448
_NBV = 28


def _ceil(a, b):
  return (a + b - 1) // b


def _tc_features(gpos_p):
  nb = _ceil(F_REAL[0] // 8, _BRF)

  def body(g0r, g1r, g2r, c1r, c2r, s6r, p35r, o_r):
    v0, v1, v2 = g0r[0], g1r[0], g2r[0]
    e1 = v1 - v0
    e2 = v2 - v0
    a1 = jnp.dot(e1, c1r[...], precision=_HI)
    a2 = jnp.dot(e1, c2r[...], precision=_HI)
    b1_ = jnp.dot(e2, c1r[...], precision=_HI)
    b2_ = jnp.dot(e2, c2r[...], precision=_HI)
    n = a1 * b2_ - a2 * b1_
    nrm = jnp.sqrt(jnp.dot(n * n, s6r[...], precision=_HI))
    inv = 1.0 / (nrm + 1e-8)
    nshift = jnp.dot(n * inv, p35r[...], precision=_HI)
    cent = (v0 + v1 + v2) * (1.0 / 3.0)
    lane = lax.broadcasted_iota(jnp.int32, (_BRF, 128), 1)
    lanem = lane % 16
    fid = ((pl.program_id(0) * _BRF
            + lax.broadcasted_iota(jnp.int32, (_BRF, 128), 0)) * 8
           + lane // 16)
    res = jnp.where(lanem < 3, cent, 0.0) + nshift
    res = res + jnp.where(lanem == 6, 0.5 * nrm, 0.0)
    res = res + jnp.where(lanem == 7, 1.0, 0.0)
    for li, thr in ((8, F_REAL[1]), (9, F_REAL[2]), (10, F_REAL[3])):
      res = res + jnp.where((lanem == li) & (fid < thr), 1.0, 0.0)
    o_r[...] = res

  gspec = lambda j: pl.BlockSpec((1, _BRF, 128), lambda i, j=j: (j, i, 0))
  cspec = pl.BlockSpec((128, 128), lambda i: (0, 0))
  return pl.pallas_call(
      body,
      grid=(nb,),
      in_specs=[gspec(0), gspec(1), gspec(2), cspec, cspec, cspec, cspec],
      out_specs=pl.BlockSpec((_BRF, 128), lambda i: (i, 0)),
      out_shape=jax.ShapeDtypeStruct((F_PAD[0] // 8, 128), jnp.float32),
  )(gpos_p, gpos_p, gpos_p, jnp.asarray(_C1), jnp.asarray(_C2),
    jnp.asarray(_S6), jnp.asarray(_P35))


def _tc_vertex1(sums1_p, b7):
  def body(sa_r, sb_r, b7r, o_r):
    s = sa_r[0] + sb_r[0]
    ivb = 1.0 / jnp.maximum(jnp.dot(s, b7r[...]), 1.0)
    o_r[...] = s * ivb

  pspec = lambda p: pl.BlockSpec((1, _BRV, 128), lambda i, p=p: (p, i, 0))
  return pl.pallas_call(
      body,
      grid=(_NBV,),
      in_specs=[pspec(0), pspec(1), pl.BlockSpec((128, 128), lambda i: (0, 0))],
      out_specs=pl.BlockSpec((_BRV, 128), lambda i: (i, 0)),
      out_shape=jax.ShapeDtypeStruct((VPAD // 8, 128), jnp.float32),
  )(sums1_p, sums1_p, b7)


def _tc_vertex(sums_p, sums1_p, wstack, bl, dout):
  lo = 8 * dout

  def body(s_r, w_r, sa_r, sb_r, bl_r, o_r):
    ci = pl.program_id(1)
    p = jnp.dot(s_r[0], w_r[0], preferred_element_type=jnp.float32)
    acc = jnp.where(ci == 0, p, o_r[...] + p)
    s1 = sa_r[0] + sb_r[0]
    ivb = 1.0 / jnp.maximum(jnp.dot(s1, bl_r[...]), 1.0)
    o_r[...] = jnp.where(ci == 3, acc * ivb, acc)

  p1spec = lambda p: pl.BlockSpec((1, _BRV, 128), lambda i, c, p=p: (p, i, 0))
  return pl.pallas_call(
      body,
      grid=(_NBV, 4),
      in_specs=[
          pl.BlockSpec((1, _BRV, 128), lambda i, c: (c, i, 0)),
          pl.BlockSpec((1, 128, lo), lambda i, c: (c, 0, 0)),
          p1spec(0), p1spec(1),
          pl.BlockSpec((128, lo), lambda i, c: (0, 0)),
      ],
      out_specs=pl.BlockSpec((_BRV, lo), lambda i, c: (i, 0)),
      out_shape=jax.ShapeDtypeStruct((VPAD // 8, lo), jnp.float32),
  )(sums_p, wstack, sums1_p, sums1_p, bl)


def _tc_face(feats_p, g_p, bdws, bt, rows, lf, lg, lo, rows_out,
             bdwn=None, relu=True):
  nb = _ceil(rows, _BRF)

  def body(fl, g0l, g1l, g2l, w_r, *rest):
    if bdwn is not None:
      wn_r = rest[0]
      rest = rest[1:]
    b_r, o_r = rest
    agg = (g0l[0] + g1l[0] + g2l[0]) * (1.0 / 3.0)
    z = jnp.dot(fl[...], w_r[...], preferred_element_type=jnp.float32)
    if bdwn is not None:
      z = z + jnp.dot(agg, wn_r[...], preferred_element_type=jnp.float32)
    else:
      z = z + agg
    z = z + b_r[...]
    o_r[...] = jnp.maximum(z, 0.0) if relu else z

  in_specs = [pl.BlockSpec((_BRF, lf), lambda i: (i, 0))]
  args = [feats_p]
  in_specs += [pl.BlockSpec((1, _BRF, lg), lambda i, j=j: (j, i, 0))
               for j in range(3)]
  args += [g_p, g_p, g_p]
  in_specs.append(pl.BlockSpec((lf, lo), lambda i: (0, 0)))
  args.append(bdws)
  if bdwn is not None:
    in_specs.append(pl.BlockSpec((lg, lo), lambda i: (0, 0)))
    args.append(bdwn)
  in_specs.append(pl.BlockSpec((1, lo), lambda i: (0, 0)))
  args.append(bt.reshape(1, lo))

  return pl.pallas_call(
      body,
      grid=(nb,),
      in_specs=in_specs,
      out_specs=pl.BlockSpec((_BRF, lo), lambda i: (i, 0)),
      out_shape=jax.ShapeDtypeStruct((rows_out, lo), jnp.float32),
  )(*args)


def _tc_pool(zlo, zhi, rows_out):
  nb = _ceil(zlo.shape[0], _BRF)

  def body(a_r, b_r, o_r):
    o_r[...] = 0.5 * (a_r[...] + b_r[...])

  spec = pl.BlockSpec((_BRF, 128), lambda i: (i, 0))
  return pl.pallas_call(
      body,
      grid=(nb,),
      in_specs=[spec, spec],
      out_specs=spec,
      out_shape=jax.ShapeDtypeStruct((rows_out, 128), jnp.float32),
  )(zlo, zhi)


def _kron8(w):
  return jnp.kron(jnp.eye(8, dtype=jnp.float32), w)


def _kron2(w):
  return jnp.kron(jnp.eye(2, dtype=jnp.float32), w)


def kernel(pos, faces, W1s, W1n, b1, W2s, W2n, b2, W3s, W3n, b3,
           W4s, W4n, b4):
  facesT = faces.T.astype(jnp.int32)
  idx = []
  for l in range(4):
    a = jnp.concatenate(
        [facesT[:, :F_REAL[l]],
         jnp.full((3, F_PAD[l] - F_REAL[l]), DUMMY, jnp.int32)], axis=1)
    idx.append(a.reshape(3, F_PAD[l] // 128, 128))

  pos16 = jnp.pad(pos.astype(jnp.float32), ((0, VPAD - V), (0, 13)))
  w1s16 = jnp.pad(W1s, ((0, 9), (0, 0)))
  w1n16 = jnp.pad(W1n, ((0, 9), (0, 0)))
  b7 = jnp.asarray(_bcast_mat(7, 16))
  bl2 = jnp.asarray(_bcast_mat(8, 64))
  bl3 = jnp.asarray(_bcast_mat(9, 64))
  bl4 = jnp.asarray(_bcast_mat(10, 16))
  wn2 = jnp.stack([_kron8(W2n[16 * c:16 * c + 16]) for c in range(4)])
  wn3 = jnp.stack([_kron8(W3n[16 * c:16 * c + 16]) for c in range(4)])
  wn4 = jnp.stack([_kron8(W4n[16 * c:16 * c + 16]) for c in range(4)])

  gpos = _sc_gather(pos16, idx[0], F_PAD[0], 16)
  feats1p = _tc_features(gpos.reshape(3, F_PAD[0] // 8, 128))
  sums1 = _sc_scatter(feats1p.reshape(F_PAD[0], 16), idx[0], F_PAD[0], 1,
                      split=True)
  sums1p = sums1.reshape(2, VPAD // 8, 128)
  vt1p = _tc_vertex1(sums1p, b7)
  g1 = _sc_gather(vt1p.reshape(VPAD, 16), idx[0], F_PAD[0], 16)
  z1 = _tc_face(feats1p, g1.reshape(3, F_PAD[0] // 8, 128),
                _kron8(w1s16), jnp.tile(b1, 8), F_REAL[0] // 8,
                128, 128, 512, F_PAD[0] // 8, bdwn=_kron8(w1n16), relu=True)
  z1p2 = z1.reshape(F_PAD[0] // 2, 128)
  feats2p = _tc_pool(z1p2[:F_REAL[0] // 4],
                     z1p2[F_REAL[0] // 4:F_REAL[0] // 2], F_PAD[1] // 2)
  feats2 = feats2p.reshape(F_PAD[1], 64)

  sums2 = _sc_scatter(feats2, idx[1], F_PAD[1], 4, split=False)
  vt2p = _tc_vertex(sums2.reshape(4, VPAD // 8, 128), sums1p, wn2, bl2, 64)
  g2 = _sc_gather(vt2p.reshape(VPAD, 64), idx[1], F_PAD[1], 64)
  z2 = _tc_face(feats2p, g2.reshape(3, F_PAD[1] // 2, 128),
                _kron2(W2s), jnp.tile(b2, 2), F_REAL[1] // 2,
                128, 128, 128, F_PAD[1] // 2, relu=True)
  feats3p = _tc_pool(z2[:F_REAL[1] // 4],
                     z2[F_REAL[1] // 4:F_REAL[1] // 2], F_PAD[2] // 2)
  feats3 = feats3p.reshape(F_PAD[2], 64)

  sums3 = _sc_scatter(feats3, idx[2], F_PAD[2], 4, split=False)
  vt3p = _tc_vertex(sums3.reshape(4, VPAD // 8, 128), sums1p, wn3, bl3, 64)
  g3 = _sc_gather(vt3p.reshape(VPAD, 64), idx[2], F_PAD[2], 64)
  z3 = _tc_face(feats3p, g3.reshape(3, F_PAD[2] // 2, 128),
                _kron2(W3s), jnp.tile(b3, 2), F_REAL[2] // 2,
                128, 128, 128, F_PAD[2] // 2, relu=True)
  feats4p = _tc_pool(z3[:F_REAL[2] // 4],
                     z3[F_REAL[2] // 4:F_REAL[2] // 2], F_PAD[3] // 2)
  feats4 = feats4p.reshape(F_PAD[3], 64)

  sums4 = _sc_scatter(feats4, idx[3], F_PAD[3], 4, split=False)
  vt4p = _tc_vertex(sums4.reshape(4, VPAD // 8, 128), sums1p, wn4, bl4, 16)
  g4 = _sc_gather(vt4p.reshape(VPAD, 16), idx[3], F_PAD[3], 16)
  outp = _tc_face(feats4.reshape(F_PAD[3] // 8, 512),
                  g4.reshape(3, F_PAD[3] // 8, 128),
                  _kron8(W4s), jnp.tile(b4, 8), F_REAL[3] // 8,
                  512, 128, 128, F_REAL[3] // 8, relu=False)

  return (faces[:F_REAL[3]], outp.reshape(F_REAL[3], 16))


# speedup vs baseline: 9.2485x; 1.0211x over previous
import functools

import jax
import jax.numpy as jnp
from jax import lax
from jax.experimental import pallas as pl
from jax.experimental.pallas import tpu as pltpu
from jax.experimental.pallas import tpu_sc as plsc

V = 100000
VPAD = 100352
DUMMY = 100000
STRIPE = VPAD // 16
F_REAL = (200000, 100000, 50000, 25000)
F_PAD = (200704, 100352, 50176, 25600)
NC, NS = 2, 16
NW = NC * NS
BF = 1000
BV = 512

_SC_PARAMS = pltpu.CompilerParams(use_tc_tiling_on_sc=False)

_mesh = functools.partial(
    plsc.VectorSubcoreMesh,
    core_axis_name="c", subcore_axis_name="s", num_cores=NC, num_subcores=NS)


def _sc_gather(table, idx3, fp, d):
  br = 1024 if d == 16 else 512
  nr = br // 128
  m_tot = fp // br

  def body(table_h, idx_h, out_h, ibuf, rows, gsem, osem):
    core = lax.axis_index("c")
    sub = lax.axis_index("s")
    wid = sub * NC + core
    cnt = (m_tot - wid + NW - 1) // NW
    for j in range(3):
      def gbody(g, _):
        for b in range(2):
          m = g * 2 + b
          @pl.when(m < cnt)
          def _():
            mb = wid + m * NW
            base = mb * br
            row = mb * nr
            @pl.when(m >= 2)
            def _():
              pltpu.make_async_copy(out_h.at[j, pl.ds(0, br), :],
                                    rows.at[b], osem).wait()
            pltpu.sync_copy(idx_h.at[j, pl.ds(row, nr), :], ibuf.at[b])
            cps = [pltpu.async_copy(table_h.at[ibuf.at[b, r]],
                                    rows.at[b, pl.ds(128 * r, 128)], gsem)
                   for r in range(nr)]
            for cp in cps:
              cp.wait()
            pltpu.async_copy(rows.at[b], out_h.at[j, pl.ds(base, br), :], osem)
        return 0
      lax.fori_loop(0, (cnt + 1) // 2, gbody, 0)
      for b in range(2):
        @pl.when(cnt >= b + 1)
        def _():
          pltpu.make_async_copy(out_h.at[j, pl.ds(0, br), :],
                                rows.at[b], osem).wait()

  return pl.kernel(
      body,
      out_type=jax.ShapeDtypeStruct((3, fp, d), jnp.float32),
      mesh=_mesh(),
      scratch_types=[
          pltpu.VMEM((2, nr, 128), jnp.int32),
          pltpu.VMEM((2, br, d), jnp.float32),
          pltpu.SemaphoreType.DMA,
          pltpu.SemaphoreType.DMA,
      ],
      compiler_params=_SC_PARAMS,
  )(table, idx3)


def _sc_scatter(data, idx3, fp, nchunks, split):
  n_out = 2 if split else nchunks
  cpc = 1 if split else nchunks // 2
  m_tot = fp // 1024
  full = split

  def body(data_h, idx_h, out_h, acc, dbuf, ib0, ib1, ib2, sem):
    core = lax.axis_index("c")
    sub = lax.axis_index("s")

    for cc in range(cpc):
      def zb(i, _):
        dbuf[i] = jnp.zeros((16,), jnp.float32)
        return 0
      lax.fori_loop(0, 1024, zb, 0)
      for t in range(6):
        pltpu.sync_copy(dbuf, acc.at[pl.ds(sub * STRIPE + t * 1024, 1024), :])
      pltpu.sync_copy(dbuf.at[pl.ds(0, 128)],
                      acc.at[pl.ds(sub * STRIPE + 6144, 128), :])
      plsc.subcore_barrier()

      if split:
        ch = 0
        m2 = m_tot // 2
        mb0 = core * m2 + sub
        cnt = (m2 - sub + NS - 1) // NS
      else:
        ch = core * cpc + cc
        mb0 = sub
        cnt = (m_tot - sub + NS - 1) // NS
      col = 16 * ch

      def mbody(m, _):
        mb = mb0 + m * NS
        base = mb * 1024
        row = mb * 8
        if full:
          pltpu.sync_copy(data_h.at[pl.ds(base, 1024), :], dbuf)
        else:
          pltpu.sync_copy(data_h.at[pl.ds(base, 1024), pl.ds(col, 16)], dbuf)
        pltpu.sync_copy(idx_h.at[0, pl.ds(row, 8), :], ib0)
        pltpu.sync_copy(idx_h.at[1, pl.ds(row, 8), :], ib1)
        pltpu.sync_copy(idx_h.at[2, pl.ds(row, 8), :], ib2)
        cps = [pltpu.async_copy(dbuf.at[pl.ds(128 * r, 128)],
                                acc.at[ib.at[r]], sem, add=True)
               for ib in (ib0, ib1, ib2) for r in range(8)]
        for cp in cps:
          cp.wait()
        return 0
      lax.fori_loop(0, cnt, mbody, 0)
      plsc.subcore_barrier()

      oi = core if split else ch
      pltpu.sync_copy(acc.at[pl.ds(sub * STRIPE, STRIPE), :],
                      out_h.at[oi, pl.ds(sub * STRIPE, STRIPE), :])
      plsc.subcore_barrier()

  return pl.kernel(
      body,
      out_type=jax.ShapeDtypeStruct((n_out, VPAD, 16), jnp.float32),
      mesh=_mesh(),
      scratch_types=[
          pltpu.VMEM_SHARED((VPAD, 16), jnp.float32),
          pltpu.VMEM((1024, 16), jnp.float32),
          pltpu.VMEM((8, 128), jnp.int32),
          pltpu.VMEM((8, 128), jnp.int32),
          pltpu.VMEM((8, 128), jnp.int32),
          pltpu.SemaphoreType.DMA,
      ],
      compiler_params=_SC_PARAMS,
  )(data, idx3)


_HI = lax.Precision.HIGHEST


def _sel(entries):
  import numpy as np
  m = np.zeros((128, 128), np.float32)
  for r, c in entries:
    m[r, c] = 1.0
  return m


_C1 = _sel([(16 * s + (k + 1) % 3, 16 * s + k)
            for s in range(8) for k in range(3)])
_C2 = _sel([(16 * s + (k + 2) % 3, 16 * s + k)
            for s in range(8) for k in range(3)])
_S6 = _sel([(16 * s + k, 16 * s + j)
            for s in range(8) for k in range(3) for j in range(7)])
_P35 = _sel([(16 * s + k, 16 * s + 3 + k)
             for s in range(8) for k in range(3)])


def _bcast_mat(cl, dout):
  import numpy as np
  m = np.zeros((128, 8 * dout), np.float32)
  for q in range(8):
    for j in range(dout):
      m[16 * q + cl, dout * q + j] = 1.0
  return m


_BRF = 512
_BRV = 448
_NBV = 28


def _ceil(a, b):
  return (a + b - 1) // b


def _tc_features(gpos_p):
  nb = _ceil(F_REAL[0] // 8, _BRF)

  def body(g0r, g1r, g2r, c1r, c2r, s6r, p35r, o_r):
    v0, v1, v2 = g0r[0], g1r[0], g2r[0]
    e1 = v1 - v0
    e2 = v2 - v0
    a1 = jnp.dot(e1, c1r[...], precision=_HI)
    a2 = jnp.dot(e1, c2r[...], precision=_HI)
    b1_ = jnp.dot(e2, c1r[...], precision=_HI)
    b2_ = jnp.dot(e2, c2r[...], precision=_HI)
    n = a1 * b2_ - a2 * b1_
    nrm = jnp.sqrt(jnp.dot(n * n, s6r[...], precision=_HI))
    inv = 1.0 / (nrm + 1e-8)
    nshift = jnp.dot(n * inv, p35r[...], precision=_HI)
    cent = (v0 + v1 + v2) * (1.0 / 3.0)
    lane = lax.broadcasted_iota(jnp.int32, (_BRF, 128), 1)
    lanem = lane % 16
    fid = ((pl.program_id(0) * _BRF
            + lax.broadcasted_iota(jnp.int32, (_BRF, 128), 0)) * 8
           + lane // 16)
    res = jnp.where(lanem < 3, cent, 0.0) + nshift
    res = res + jnp.where(lanem == 6, 0.5 * nrm, 0.0)
    res = res + jnp.where(lanem == 7, 1.0, 0.0)
    for li, thr in ((8, F_REAL[1]), (9, F_REAL[2]), (10, F_REAL[3])):
      res = res + jnp.where((lanem == li) & (fid < thr), 1.0, 0.0)
    o_r[...] = res

  gspec = lambda j: pl.BlockSpec((1, _BRF, 128), lambda i, j=j: (j, i, 0))
  cspec = pl.BlockSpec((128, 128), lambda i: (0, 0))
  return pl.pallas_call(
      body,
      grid=(nb,),
      in_specs=[gspec(0), gspec(1), gspec(2), cspec, cspec, cspec, cspec],
      out_specs=pl.BlockSpec((_BRF, 128), lambda i: (i, 0)),
      out_shape=jax.ShapeDtypeStruct((F_PAD[0] // 8, 128), jnp.float32),
  )(gpos_p, gpos_p, gpos_p, jnp.asarray(_C1), jnp.asarray(_C2),
    jnp.asarray(_S6), jnp.asarray(_P35))


def _tc_vertex1(sums1_p, b7):
  def body(sa_r, sb_r, b7r, o_r):
    s = sa_r[0] + sb_r[0]
    ivb = 1.0 / jnp.maximum(jnp.dot(s, b7r[...]), 1.0)
    o_r[...] = s * ivb

  pspec = lambda p: pl.BlockSpec((1, _BRV, 128), lambda i, p=p: (p, i, 0))
  return pl.pallas_call(
      body,
      grid=(_NBV,),
      in_specs=[pspec(0), pspec(1), pl.BlockSpec((128, 128), lambda i: (0, 0))],
      out_specs=pl.BlockSpec((_BRV, 128), lambda i: (i, 0)),
      out_shape=jax.ShapeDtypeStruct((VPAD // 8, 128), jnp.float32),
  )(sums1_p, sums1_p, b7)


def _tc_vertex(sums_p, sums1_p, wstack, bl, dout):
  lo = 8 * dout

  def body(s_r, w_r, sa_r, sb_r, bl_r, o_r):
    ci = pl.program_id(1)
    p = jnp.dot(s_r[0], w_r[0], preferred_element_type=jnp.float32)
    acc = jnp.where(ci == 0, p, o_r[...] + p)
    s1 = sa_r[0] + sb_r[0]
    ivb = 1.0 / jnp.maximum(jnp.dot(s1, bl_r[...]), 1.0)
    o_r[...] = jnp.where(ci == 3, acc * ivb, acc)

  p1spec = lambda p: pl.BlockSpec((1, _BRV, 128), lambda i, c, p=p: (p, i, 0))
  return pl.pallas_call(
      body,
      grid=(_NBV, 4),
      in_specs=[
          pl.BlockSpec((1, _BRV, 128), lambda i, c: (c, i, 0)),
          pl.BlockSpec((1, 128, lo), lambda i, c: (c, 0, 0)),
          p1spec(0), p1spec(1),
          pl.BlockSpec((128, lo), lambda i, c: (0, 0)),
      ],
      out_specs=pl.BlockSpec((_BRV, lo), lambda i, c: (i, 0)),
      out_shape=jax.ShapeDtypeStruct((VPAD // 8, lo), jnp.float32),
  )(sums_p, wstack, sums1_p, sums1_p, bl)


def _tc_face(feats_p, g_p, bdws, bt, rows, lf, lg, lo, rows_out,
             bdwn=None, relu=True):
  nb = _ceil(rows, _BRF)

  def body(fl, g0l, g1l, g2l, w_r, *rest):
    if bdwn is not None:
      wn_r = rest[0]
      rest = rest[1:]
    b_r, o_r = rest
    agg = (g0l[0] + g1l[0] + g2l[0]) * (1.0 / 3.0)
    z = jnp.dot(fl[...], w_r[...], preferred_element_type=jnp.float32)
    if bdwn is not None:
      z = z + jnp.dot(agg, wn_r[...], preferred_element_type=jnp.float32)
    else:
      z = z + agg
    z = z + b_r[...]
    o_r[...] = jnp.maximum(z, 0.0) if relu else z

  in_specs = [pl.BlockSpec((_BRF, lf), lambda i: (i, 0))]
  args = [feats_p]
  in_specs += [pl.BlockSpec((1, _BRF, lg), lambda i, j=j: (j, i, 0))
               for j in range(3)]
  args += [g_p, g_p, g_p]
  in_specs.append(pl.BlockSpec((lf, lo), lambda i: (0, 0)))
  args.append(bdws)
  if bdwn is not None:
    in_specs.append(pl.BlockSpec((lg, lo), lambda i: (0, 0)))
    args.append(bdwn)
  in_specs.append(pl.BlockSpec((1, lo), lambda i: (0, 0)))
  args.append(bt.reshape(1, lo))

  return pl.pallas_call(
      body,
      grid=(nb,),
      in_specs=in_specs,
      out_specs=pl.BlockSpec((_BRF, lo), lambda i: (i, 0)),
      out_shape=jax.ShapeDtypeStruct((rows_out, lo), jnp.float32),
  )(*args)


def _tc_pool(zlo, zhi, rows_out):
  nb = _ceil(zlo.shape[0], _BRF)

  def body(a_r, b_r, o_r):
    o_r[...] = 0.5 * (a_r[...] + b_r[...])

  spec = pl.BlockSpec((_BRF, 128), lambda i: (i, 0))
  return pl.pallas_call(
      body,
      grid=(nb,),
      in_specs=[spec, spec],
      out_specs=spec,
      out_shape=jax.ShapeDtypeStruct((rows_out, 128), jnp.float32),
  )(zlo, zhi)


def _kron8(w):
  return jnp.kron(jnp.eye(8, dtype=jnp.float32), w)


def _kron2(w):
  return jnp.kron(jnp.eye(2, dtype=jnp.float32), w)


def kernel(pos, faces, W1s, W1n, b1, W2s, W2n, b2, W3s, W3n, b3,
           W4s, W4n, b4):
  facesT = faces.T.astype(jnp.int32)
  idx = []
  for l in range(4):
    a = jnp.concatenate(
        [facesT[:, :F_REAL[l]],
         jnp.full((3, F_PAD[l] - F_REAL[l]), DUMMY, jnp.int32)], axis=1)
    idx.append(a.reshape(3, F_PAD[l] // 128, 128))

  pos16 = jnp.pad(pos.astype(jnp.float32), ((0, VPAD - V), (0, 13)))
  w1s16 = jnp.pad(W1s, ((0, 9), (0, 0)))
  w1n16 = jnp.pad(W1n, ((0, 9), (0, 0)))
  b7 = jnp.asarray(_bcast_mat(7, 16))
  bl2 = jnp.asarray(_bcast_mat(8, 64))
  bl3 = jnp.asarray(_bcast_mat(9, 64))
  bl4 = jnp.asarray(_bcast_mat(10, 16))
  wn2 = jnp.stack([_kron8(W2n[16 * c:16 * c + 16]) for c in range(4)])
  wn3 = jnp.stack([_kron8(W3n[16 * c:16 * c + 16]) for c in range(4)])
  wn4 = jnp.stack([_kron8(W4n[16 * c:16 * c + 16]) for c in range(4)])

  gpos = _sc_gather(pos16, idx[0], F_PAD[0], 16)
  feats1p = _tc_features(gpos.reshape(3, F_PAD[0] // 8, 128))
  sums1 = _sc_scatter(feats1p.reshape(F_PAD[0], 16), idx[0], F_PAD[0], 1,
                      split=True)
  sums1p = sums1.reshape(2, VPAD // 8, 128)
  vt1p = _tc_vertex1(sums1p, b7)
  g1 = _sc_gather(vt1p.reshape(VPAD, 16), idx[0], F_PAD[0], 16)
  z1 = _tc_face(feats1p, g1.reshape(3, F_PAD[0] // 8, 128),
                _kron8(w1s16), jnp.tile(b1, 8), F_REAL[0] // 8,
                128, 128, 512, F_PAD[0] // 8, bdwn=_kron8(w1n16), relu=True)
  z1p2 = z1.reshape(F_PAD[0] // 2, 128)
  feats2p = _tc_pool(z1p2[:F_REAL[0] // 4],
                     z1p2[F_REAL[0] // 4:F_REAL[0] // 2], F_PAD[1] // 2)
  feats2 = feats2p.reshape(F_PAD[1], 64)

  sums2 = _sc_scatter(feats2, idx[1], F_PAD[1], 4, split=False)
  vt2p = _tc_vertex(sums2.reshape(4, VPAD // 8, 128), sums1p, wn2, bl2, 64)
  g2 = _sc_gather(vt2p.reshape(VPAD, 64), idx[1], F_PAD[1], 64)
  z2 = _tc_face(feats2p, g2.reshape(3, F_PAD[1] // 2, 128),
                _kron2(W2s), jnp.tile(b2, 2), F_REAL[1] // 2,
                128, 128, 128, F_PAD[1] // 2, relu=True)
  feats3p = _tc_pool(z2[:F_REAL[1] // 4],
                     z2[F_REAL[1] // 4:F_REAL[1] // 2], F_PAD[2] // 2)
  feats3 = feats3p.reshape(F_PAD[2], 64)

  sums3 = _sc_scatter(feats3, idx[2], F_PAD[2], 4, split=False)
  vt3p = _tc_vertex(sums3.reshape(4, VPAD // 8, 128), sums1p, wn3, bl3, 64)
  g3 = _sc_gather(vt3p.reshape(VPAD, 64), idx[2], F_PAD[2], 64)
  z3 = _tc_face(feats3p, g3.reshape(3, F_PAD[2] // 2, 128),
                _kron2(W3s), jnp.tile(b3, 2), F_REAL[2] // 2,
                128, 128, 128, F_PAD[2] // 2, relu=True)
  feats4p = _tc_pool(z3[:F_REAL[2] // 4],
                     z3[F_REAL[2] // 4:F_REAL[2] // 2], F_PAD[3] // 2)
  feats4 = feats4p.reshape(F_PAD[3], 64)

  sums4 = _sc_scatter(feats4, idx[3], F_PAD[3], 4, split=False)
  vt4p = _tc_vertex(sums4.reshape(4, VPAD // 8, 128), sums1p, wn4, bl4, 16)
  g4 = _sc_gather(vt4p.reshape(VPAD, 16), idx[3], F_PAD[3], 16)
  outp = _tc_face(feats4.reshape(F_PAD[3] // 8, 512),
                  g4.reshape(3, F_PAD[3] // 8, 128),
                  _kron8(W4s), jnp.tile(b4, 8), F_REAL[3] // 8,
                  512, 128, 128, F_REAL[3] // 8, relu=False)

  return (faces[:F_REAL[3]], outp.reshape(F_REAL[3], 16))
